# Initial kernel scaffold; baseline (speedup 1.0000x reference)
import jax
import jax.numpy as jnp
from jax.experimental import pallas as pl


def kernel(x, W, bias):
    raise NotImplementedError("write your pallas kernel here")


# speedup vs baseline: 1.5154x; 1.5154x over previous
import functools

import jax
import jax.numpy as jnp
from jax.experimental import pallas as pl
from jax.experimental.pallas import tpu as pltpu

_N_EXPERTS = 16
_TOPK = 2
_BT = 1024


def _router_body(x_ref, wt_ref, bias_ref, w_out_ref, idx_out_ref):
    logits = jnp.dot(x_ref[...], wt_ref[...], preferred_element_type=jnp.float32)
    scores = jax.nn.sigmoid(logits)
    s_sel = scores + bias_ref[...]

    lane = jax.lax.broadcasted_iota(jnp.int32, s_sel.shape, 1)

    m1 = jnp.max(s_sel, axis=1, keepdims=True)
    i1 = jnp.min(jnp.where(s_sel == m1, lane, _N_EXPERTS), axis=1, keepdims=True)
    masked = jnp.where(lane == i1, -jnp.inf, s_sel)
    m2 = jnp.max(masked, axis=1, keepdims=True)
    i2 = jnp.min(jnp.where(masked == m2, lane, _N_EXPERTS), axis=1, keepdims=True)

    w1 = jnp.sum(jnp.where(lane == i1, scores, 0.0), axis=1, keepdims=True)
    w2 = jnp.sum(jnp.where(lane == i2, scores, 0.0), axis=1, keepdims=True)
    denom = jnp.maximum(w1 + w2, 1e-12)

    w_out_ref[...] = jnp.concatenate([w1 / denom, w2 / denom], axis=1)
    idx_out_ref[...] = jnp.concatenate([i1, i2], axis=1)


@jax.jit
def kernel(x, W, bias):
    tokens, dim = x.shape
    n_experts = W.shape[0]
    wt = W.T
    bias_row = bias[None, :]
    grid = (tokens // _BT,)
    weights, indices = pl.pallas_call(
        _router_body,
        grid=grid,
        in_specs=[
            pl.BlockSpec((_BT, dim), lambda i: (i, 0)),
            pl.BlockSpec((dim, n_experts), lambda i: (0, 0)),
            pl.BlockSpec((1, n_experts), lambda i: (0, 0)),
        ],
        out_specs=[
            pl.BlockSpec((_BT, _TOPK), lambda i: (i, 0)),
            pl.BlockSpec((_BT, _TOPK), lambda i: (i, 0)),
        ],
        out_shape=[
            jax.ShapeDtypeStruct((tokens, _TOPK), jnp.float32),
            jax.ShapeDtypeStruct((tokens, _TOPK), jnp.int32),
        ],
    )(x, wt, bias_row)
    return weights, indices
